# baseline (speedup 1.0000x reference)
import jax
import jax.numpy as jnp
from jax.experimental import pallas as pl
from jax.experimental.pallas import tpu as pltpu

_B = 4
_T = 14
_TP = 16


def _zero_pads(ref, n_phase, c):
    def body(p, carry):
        ref[p, :, _T, :, :] = jnp.zeros((_B, _TP, c), jnp.float32)
        ref[p, :, :, _T, :] = jnp.zeros((_B, _TP, c), jnp.float32)
        return carry

    jax.lax.fori_loop(0, n_phase, body, 0)


def _conv_phase_split(in_ref, w_ref, s_in, out_ref):
    s_out = s_in // 2

    def tap_acc(er, ec, er_last, ec_last):
        acc = None
        for dy in range(3):
            ar = 1 if (er_last and dy == 2) else 0
            pr = 2 * er + dy - s_in * ar
            for dx in range(3):
                ac = 1 if (ec_last and dx == 2) else 0
                pc = 2 * ec + dx - s_in * ac
                sl = in_ref[pr * s_in + pc, :, ar:ar + _T, :, :]
                if ac:
                    sl = jnp.roll(sl, -1, axis=2)
                d = jax.lax.dot_general(
                    sl, w_ref[dy * 3 + dx],
                    (((3,), (0,)), ((), ())),
                    preferred_element_type=jnp.float32)
                acc = d if acc is None else acc + d
        return jnp.maximum(acc, 0.0)

    m = s_out - 1

    def interior(i, carry):
        er, ec = i // m, i % m
        out_ref[er * s_out + ec, :, 0:_T, 0:_T, :] = tap_acc(
            er, ec, False, False)[:, :, 0:_T, :]
        return carry

    def bottom(ec, carry):
        out_ref[m * s_out + ec, :, 0:_T, 0:_T, :] = tap_acc(
            m, ec, True, False)[:, :, 0:_T, :]
        return carry

    def right(er, carry):
        out_ref[er * s_out + m, :, 0:_T, 0:_T, :] = tap_acc(
            er, m, False, True)[:, :, 0:_T, :]
        return carry

    if m > 0:
        jax.lax.fori_loop(0, m * m, interior, 0)
        jax.lax.fori_loop(0, m, bottom, 0)
        jax.lax.fori_loop(0, m, right, 0)
    val = tap_acc(m, m, True, True)
    if out_ref is None:
        return val
    out_ref[m * s_out + m, :, 0:_T, 0:_T, :] = val[:, :, 0:_T, :]
    return None


def _body(patch_ref, w1_ref, w2_ref, w3_ref, w4_ref, kT_ref, keys_ref,
          out_ref, s1, s2, s3):
    _zero_pads(s1, 64, 32)
    _zero_pads(s2, 16, 64)
    _zero_pads(s3, 4, 128)

    w1 = w1_ref[...]

    def l1_body(p, carry):
        sl = patch_ref[p].reshape(_B, _T, _TP, 27)
        d = jax.lax.dot_general(sl, w1, (((3,), (0,)), ((), ())),
                                preferred_element_type=jnp.float32)
        s1[p, :, 0:_T, 0:_T, :] = jnp.maximum(d[:, :, 0:_T, :], 0.0)
        return carry

    jax.lax.fori_loop(0, 64, l1_body, 0)

    _conv_phase_split(s1, w2_ref, 8, s2)
    _conv_phase_split(s2, w3_ref, 4, s3)
    fea = _conv_phase_split(s3, w4_ref, 2, None)

    n2 = jnp.sum(fea * fea, axis=-1, keepdims=True)
    q = fea / jnp.maximum(jnp.sqrt(n2), 1e-12)

    kT = kT_ref[...]
    keys = keys_ref[...]
    scores = jax.lax.dot_general(
        q, kT, (((3,), (0,)), ((), ())),
        preferred_element_type=jnp.float32)
    ksq = jnp.sum(kT * kT, axis=0)
    adj = scores - 0.5 * ksq
    ii = jax.lax.broadcasted_iota(jnp.int32, adj.shape, 3)

    def argmax_first(a):
        mx = jnp.max(a, axis=-1, keepdims=True)
        return jnp.min(jnp.where(a == mx, ii, 512), axis=-1, keepdims=True)

    i1 = argmax_first(adj)
    adj2 = jnp.where(ii == i1, -jnp.inf, adj)
    i2 = argmax_first(adj2)

    def gather_key(idx):
        oh = (ii == idx).astype(jnp.float32)
        return jax.lax.dot_general(
            oh, keys, (((3,), (0,)), ((), ())),
            preferred_element_type=jnp.float32)

    k1, k2 = gather_key(i1), gather_key(i2)
    d1 = jnp.sum((q - k1) ** 2, axis=-1, keepdims=True)
    d2 = jnp.sum((q - k2) ** 2, axis=-1, keepdims=True)
    take2 = (d2 < d1) | ((d2 == d1) & (i2 < i1))
    nk = jnp.where(take2, k2, k1)
    d = q - nk
    dd = d * d
    heat = jnp.sum(dd * dd, axis=-1)
    out_ref[...] = heat[:, :, 0:_T]


def kernel(x, W1, W2, W3, W4, keys):
    xh = jnp.transpose(x, (0, 2, 3, 1))
    xp = jnp.pad(xh, ((0, 0), (0, 2), (0, 2), (0, 0)))
    taps = []
    for dy in range(3):
        for dx in range(3):
            a = jax.lax.slice(xp, (0, dy, dx, 0), (_B, dy + 224, dx + 224, 3),
                              (1, 2, 2, 1))
            taps.append(a.reshape(_B, _T, 8, _T, 8, 3))
    pt = jnp.stack(taps, axis=5)
    pt = jnp.pad(pt, ((0, 0), (0, 0), (0, 0), (0, 2), (0, 0), (0, 0), (0, 0)))
    patches = (pt.transpose(2, 4, 0, 1, 3, 5, 6)
               .reshape(64, _B, _T, _TP * 27))

    w1 = jnp.transpose(W1, (2, 3, 1, 0)).reshape(27, 32)
    w2 = jnp.transpose(W2, (2, 3, 1, 0)).reshape(9, 32, 64)
    w3 = jnp.transpose(W3, (2, 3, 1, 0)).reshape(9, 64, 128)
    w4 = jnp.transpose(W4, (2, 3, 1, 0)).reshape(9, 128, 128)
    kT = keys.T

    out = pl.pallas_call(
        _body,
        out_shape=jax.ShapeDtypeStruct((_B, _T, _T), jnp.float32),
        scratch_shapes=[
            pltpu.VMEM((64, _B, _TP, _TP, 32), jnp.float32),
            pltpu.VMEM((16, _B, _TP, _TP, 64), jnp.float32),
            pltpu.VMEM((4, _B, _TP, _TP, 128), jnp.float32),
        ],
    )(patches, w1, w2, w3, w4, kT, keys)
    return out.reshape(_B, _T, _T, 1)


# speedup vs baseline: 5.9374x; 5.9374x over previous
import jax
import jax.numpy as jnp
from jax.experimental import pallas as pl


def _conv2d_s2(x, w):
    return jax.lax.conv_general_dilated(
        x, w, window_strides=(2, 2), padding='SAME',
        dimension_numbers=('NCHW', 'OIHW', 'NCHW'))


def _head(q_ref, kT_ref, keys_ref, out_ref):
    q = q_ref[...]
    kT = kT_ref[...]
    keys = keys_ref[...]
    scores = jax.lax.dot_general(
        q, kT, (((1,), (0,)), ((), ())),
        preferred_element_type=jnp.float32,
        precision=jax.lax.Precision.HIGHEST)
    ksq = jnp.sum(kT * kT, axis=0)
    adj = scores - 0.5 * ksq
    ii = jax.lax.broadcasted_iota(jnp.int32, adj.shape, 1)

    def argmax_first(a):
        mx = jnp.max(a, axis=-1, keepdims=True)
        return jnp.min(jnp.where(a == mx, ii, 512), axis=-1, keepdims=True)

    i1 = argmax_first(adj)
    i2 = argmax_first(jnp.where(ii == i1, -jnp.inf, adj))

    def gather_key(idx):
        oh = (ii == idx).astype(jnp.float32)
        return jax.lax.dot_general(
            oh, keys, (((1,), (0,)), ((), ())),
            preferred_element_type=jnp.float32,
            precision=jax.lax.Precision.HIGHEST)

    k1, k2 = gather_key(i1), gather_key(i2)
    d1 = jnp.sum((q - k1) ** 2, axis=-1, keepdims=True)
    d2 = jnp.sum((q - k2) ** 2, axis=-1, keepdims=True)
    take2 = (d2 < d1) | ((d2 == d1) & (i2 < i1))
    nk = jnp.where(take2, k2, k1)
    d = q - nk
    dd = d * d
    out_ref[...] = jnp.sum(dd * dd, axis=-1, keepdims=True)


def kernel(x, W1, W2, W3, W4, keys):
    h1 = jax.nn.relu(_conv2d_s2(x, W1))
    h2 = jax.nn.relu(_conv2d_s2(h1, W2))
    h3 = jax.nn.relu(_conv2d_s2(h2, W3))
    fea = jax.nn.relu(_conv2d_s2(h3, W4))
    norm = jnp.maximum(jnp.linalg.norm(fea, axis=1, keepdims=True), 1e-12)
    query = jnp.transpose(fea / norm, (0, 2, 3, 1))
    b, h, w, dch = query.shape
    q = query.reshape(b * h * w, dch)

    heat = pl.pallas_call(
        _head,
        out_shape=jax.ShapeDtypeStruct((b * h * w, 1), jnp.float32),
    )(q, keys.T, keys)
    return heat.reshape(b, h, w, 1)
